# baseline (speedup 1.0000x reference)
import jax
import jax.numpy as jnp
from jax.experimental import pallas as pl

_B = 16384
_N = 1000
_BR = 4096


def _body(x_ref, lt_ref, o_ref):
    xv = x_ref[...]
    m = jnp.max(xv, axis=1, keepdims=True)
    cols = jax.lax.broadcasted_iota(jnp.int32, xv.shape, 1)
    idx = jnp.min(jnp.where(xv == m, cols, _N), axis=1)
    tab = jnp.exp(lt_ref[...])
    onehot = cols == idx[:, None]
    tau = jnp.sum(jnp.where(onehot, tab, 0.0), axis=1)
    o_ref[...] = tau[:, None]


def kernel(x, log_tau):
    lt2 = log_tau.reshape(1, _N)
    out = pl.pallas_call(
        _body,
        grid=(_B // _BR,),
        in_specs=[
            pl.BlockSpec((_BR, _N), lambda i: (i, 0)),
            pl.BlockSpec((1, _N), lambda i: (0, 0)),
        ],
        out_specs=pl.BlockSpec((_BR, 1), lambda i: (i, 0)),
        out_shape=jax.ShapeDtypeStruct((_B, 1), jnp.float32),
    )(x, lt2)
    return out


# speedup vs baseline: 1.0020x; 1.0020x over previous
import jax
import jax.numpy as jnp
from jax.experimental import pallas as pl

_B = 16384
_N = 1000
_BRH = 1024


def _tau_rows(xv, tab):
    m = jnp.max(xv, axis=1, keepdims=True)
    cols = jax.lax.broadcasted_iota(jnp.int32, xv.shape, 1)
    idx = jnp.min(jnp.where(xv == m, cols, _N), axis=1)
    onehot = cols == idx[:, None]
    return jnp.sum(jnp.where(onehot, tab, 0.0), axis=1)


def _body(xa_ref, xb_ref, lt_ref, o_ref):
    tab = jnp.exp(lt_ref[...])
    o_ref[:_BRH, :] = _tau_rows(xa_ref[...], tab)[:, None]
    o_ref[_BRH:, :] = _tau_rows(xb_ref[...], tab)[:, None]


def kernel(x, log_tau):
    lt2 = log_tau.reshape(1, _N)
    out = pl.pallas_call(
        _body,
        grid=(_B // (2 * _BRH),),
        in_specs=[
            pl.BlockSpec((_BRH, _N), lambda i: (2 * i, 0)),
            pl.BlockSpec((_BRH, _N), lambda i: (2 * i + 1, 0)),
            pl.BlockSpec((1, _N), lambda i: (0, 0)),
        ],
        out_specs=pl.BlockSpec((2 * _BRH, 1), lambda i: (i, 0)),
        out_shape=jax.ShapeDtypeStruct((_B, 1), jnp.float32),
    )(x, x, lt2)
    return out
